# baseline (speedup 1.0000x reference)
import functools

import jax
import jax.numpy as jnp
from jax import lax
from jax.experimental import pallas as pl
from jax.experimental.pallas import tpu as pltpu
from jax.experimental.pallas import tpu_sc as plsc

ROWS = 128
COLS = 32768
WORDS = COLS // 4
NW = 32
ROWS_PER_W = ROWS // NW
F0 = 128
FB = 512
NBLK = WORDS // FB
BIG = 1 << 30

_GDN = lax.GatherDimensionNumbers(
    offset_dims=(), collapsed_slice_dims=(0,), start_index_map=(0,)
)


def _shuffle(v, perm):
    return lax.gather(
        v, perm[:, None], dimension_numbers=_GDN, slice_sizes=(1,),
        mode=lax.GatherScatterMode.PROMISE_IN_BOUNDS,
    )


def _scan_block(get_vec, num_vecs, word_base):
    lane = lax.iota(jnp.int32, 16)
    acc = jnp.full((16,), BIG, jnp.int32)
    for j in range(num_vecs):
        v = get_vec(j)
        off = jnp.where(
            (v & 0xFF) != 0, 0,
            jnp.where((v & 0xFFFF) != 0, 1, jnp.where((v & 0xFFFFFF) != 0, 2, 3)),
        ).astype(jnp.int32)
        cand = (lane + (word_base + j * 16)) * 4 + off
        acc = jnp.minimum(acc, jnp.where(v != 0, cand, BIG))
    for s in (8, 4, 2, 1):
        acc = jnp.minimum(acc, _shuffle(acc, lane ^ s))
    return acc[0]


def _make_sc_kernel():
    mesh = plsc.VectorSubcoreMesh(
        core_axis_name="c", subcore_axis_name="s", num_cores=2, num_subcores=16
    )

    @functools.partial(
        pl.kernel,
        out_type=jax.ShapeDtypeStruct((NW, 16), jnp.int32),
        mesh=mesh,
        scratch_types=[
            pltpu.VMEM((ROWS_PER_W, F0), jnp.int32),
            pltpu.VMEM((FB,), jnp.int32),
            pltpu.VMEM((16,), jnp.int32),
            pltpu.SMEM((1,), jnp.int32),
        ],
    )
    def sc_kernel(words_hbm, out_hbm, buf0, buf_fb, out_v, cur_s):
        wid = lax.axis_index("c") * 16 + lax.axis_index("s")
        row_base = wid * ROWS_PER_W

        pltpu.sync_copy(words_hbm.at[pl.ds(row_base, ROWS_PER_W), pl.ds(0, F0)], buf0)

        lane = lax.iota(jnp.int32, 16)
        res_vec = jnp.zeros((16,), jnp.int32)
        for i in range(ROWS_PER_W):
            cur_s[0] = _scan_block(lambda j: buf0[i, pl.ds(j * 16, 16)], F0 // 16, 0)

            def fb_body(blk, carry):
                @pl.when(cur_s[0] >= BIG)
                def _():
                    pltpu.sync_copy(
                        words_hbm.at[row_base + i, pl.ds(blk * FB, FB)], buf_fb
                    )
                    bmin = _scan_block(
                        lambda j: buf_fb[pl.ds(j * 16, 16)], FB // 16, blk * FB
                    )
                    cur_s[0] = jnp.minimum(cur_s[0], bmin)
                return carry

            lax.fori_loop(0, NBLK, fb_body, jnp.int32(0), unroll=False)
            s0 = cur_s[0]
            action = jnp.where(s0 >= BIG, jnp.int32(0), s0)
            res_vec = jnp.where(lane == i, action, res_vec)

        out_v[...] = res_vec
        pltpu.sync_copy(out_v, out_hbm.at[wid])

    return sc_kernel


_get_sc_kernel = functools.cache(_make_sc_kernel)


def kernel(allowed_action_mask):
    words = allowed_action_mask.view(jnp.int32)
    out = _get_sc_kernel()(words)
    return out[:, :ROWS_PER_W].reshape(ROWS)


# speedup vs baseline: 2.5970x; 2.5970x over previous
import functools

import jax
import jax.numpy as jnp
from jax import lax
from jax.experimental import pallas as pl
from jax.experimental.pallas import tpu as pltpu
from jax.experimental.pallas import tpu_sc as plsc

ROWS = 128
COLS = 32768
NW = 32
ROWS_PER_W = ROWS // NW
F0 = 512
FB = 2048
NBLK = COLS // FB
BIG = 1 << 30

_GDN = lax.GatherDimensionNumbers(
    offset_dims=(), collapsed_slice_dims=(0,), start_index_map=(0,)
)


def _shuffle(v, perm):
    return lax.gather(
        v, perm[:, None], dimension_numbers=_GDN, slice_sizes=(1,),
        mode=lax.GatherScatterMode.PROMISE_IN_BOUNDS,
    )


def _scan_block(get_vec, num_vecs, col_base):
    lane = lax.iota(jnp.int32, 16)
    acc = jnp.full((16,), BIG, jnp.int32)
    for j in range(num_vecs):
        v = get_vec(j)
        cand = lane + (col_base + j * 16)
        acc = jnp.minimum(acc, jnp.where(v != 0, cand, BIG))
    for s in (8, 4, 2, 1):
        acc = jnp.minimum(acc, _shuffle(acc, lane ^ s))
    return acc[0]


def _make_sc_kernel():
    mesh = plsc.VectorSubcoreMesh(
        core_axis_name="c", subcore_axis_name="s", num_cores=2, num_subcores=16
    )

    @functools.partial(
        pl.kernel,
        out_type=jax.ShapeDtypeStruct((NW, 16), jnp.int32),
        mesh=mesh,
        scratch_types=[
            pltpu.VMEM((ROWS_PER_W, F0), jnp.int32),
            pltpu.VMEM((FB,), jnp.int32),
            pltpu.VMEM((16,), jnp.int32),
            pltpu.SMEM((1,), jnp.int32),
        ],
    )
    def sc_kernel(mask_hbm, out_hbm, buf0, buf_fb, out_v, cur_s):
        wid = lax.axis_index("c") * 16 + lax.axis_index("s")
        row_base = wid * ROWS_PER_W

        pltpu.sync_copy(mask_hbm.at[pl.ds(row_base, ROWS_PER_W), pl.ds(0, F0)], buf0)

        lane = lax.iota(jnp.int32, 16)
        res_vec = jnp.zeros((16,), jnp.int32)
        for i in range(ROWS_PER_W):
            cur_s[0] = _scan_block(
                lambda j: buf0[i, pl.ds(j * 16, 16)], F0 // 16, 0
            )

            def fb_body(blk, carry):
                @pl.when(cur_s[0] >= BIG)
                def _():
                    pltpu.sync_copy(
                        mask_hbm.at[row_base + i, pl.ds(blk * FB, FB)], buf_fb
                    )
                    bmin = _scan_block(
                        lambda j: buf_fb[pl.ds(j * 16, 16)], FB // 16, blk * FB
                    )
                    cur_s[0] = jnp.minimum(cur_s[0], bmin)
                return carry

            lax.fori_loop(0, NBLK, fb_body, jnp.int32(0), unroll=False)
            s0 = cur_s[0]
            action = jnp.where(s0 >= BIG, jnp.int32(0), s0)
            res_vec = jnp.where(lane == i, action, res_vec)

        out_v[...] = res_vec
        pltpu.sync_copy(out_v, out_hbm.at[wid])

    return sc_kernel


_get_sc_kernel = functools.cache(_make_sc_kernel)


def kernel(allowed_action_mask):
    cols = allowed_action_mask.astype(jnp.int32)
    out = _get_sc_kernel()(cols)
    return out[:, :ROWS_PER_W].reshape(ROWS)


# speedup vs baseline: 3.2775x; 1.2620x over previous
import functools

import jax
import jax.numpy as jnp
from jax import lax
from jax.experimental import pallas as pl
from jax.experimental.pallas import tpu as pltpu
from jax.experimental.pallas import tpu_sc as plsc

ROWS = 128
COLS = 32768
NW = 32
ROWS_PER_W = ROWS // NW
F0 = 512
FB = 2048
NBLK = COLS // FB
BIG = 1 << 30

_GDN = lax.GatherDimensionNumbers(
    offset_dims=(), collapsed_slice_dims=(0,), start_index_map=(0,)
)


def _shuffle(v, perm):
    return lax.gather(
        v, perm[:, None], dimension_numbers=_GDN, slice_sizes=(1,),
        mode=lax.GatherScatterMode.PROMISE_IN_BOUNDS,
    )


def _scan_block(get_vec, num_vecs, col_base):
    lane = lax.iota(jnp.int32, 16)
    acc = jnp.full((16,), BIG, jnp.int32)
    for j in range(num_vecs):
        v = get_vec(j)
        cand = lane + (col_base + j * 16)
        acc = jnp.minimum(acc, jnp.where(v != 0, cand, BIG))
    for s in (8, 4, 2, 1):
        acc = jnp.minimum(acc, _shuffle(acc, lane ^ s))
    return acc[0]


_MESH = dict(core_axis_name="c", subcore_axis_name="s", num_cores=2, num_subcores=16)


def _make_head_kernel():
    @functools.partial(
        pl.kernel,
        out_type=jax.ShapeDtypeStruct((NW, 16), jnp.int32),
        mesh=plsc.VectorSubcoreMesh(**_MESH),
        scratch_types=[
            pltpu.VMEM((ROWS_PER_W, F0), jnp.int32),
            pltpu.VMEM((16,), jnp.int32),
        ],
    )
    def head_kernel(head_hbm, out_hbm, buf0, out_v):
        wid = lax.axis_index("c") * 16 + lax.axis_index("s")
        row_base = wid * ROWS_PER_W
        pltpu.sync_copy(head_hbm.at[pl.ds(row_base, ROWS_PER_W), pl.ds(0, F0)], buf0)

        lane = lax.iota(jnp.int32, 16)
        res_vec = jnp.zeros((16,), jnp.int32)
        for i in range(ROWS_PER_W):
            s0 = _scan_block(lambda j: buf0[i, pl.ds(j * 16, 16)], F0 // 16, 0)
            found = (s0 < BIG).astype(jnp.int32)
            action = jnp.where(s0 < BIG, s0, jnp.int32(0))
            res_vec = jnp.where(lane == i, action, res_vec)
            res_vec = jnp.where(lane == 4 + i, found, res_vec)

        out_v[...] = res_vec
        pltpu.sync_copy(out_v, out_hbm.at[wid])

    return head_kernel


def _make_full_kernel():
    @functools.partial(
        pl.kernel,
        out_type=jax.ShapeDtypeStruct((NW, 16), jnp.int32),
        mesh=plsc.VectorSubcoreMesh(**_MESH),
        scratch_types=[
            pltpu.VMEM((ROWS_PER_W, F0), jnp.int32),
            pltpu.VMEM((FB,), jnp.int32),
            pltpu.VMEM((16,), jnp.int32),
            pltpu.SMEM((1,), jnp.int32),
        ],
    )
    def full_kernel(mask_hbm, out_hbm, buf0, buf_fb, out_v, cur_s):
        wid = lax.axis_index("c") * 16 + lax.axis_index("s")
        row_base = wid * ROWS_PER_W
        pltpu.sync_copy(mask_hbm.at[pl.ds(row_base, ROWS_PER_W), pl.ds(0, F0)], buf0)

        lane = lax.iota(jnp.int32, 16)
        res_vec = jnp.zeros((16,), jnp.int32)
        for i in range(ROWS_PER_W):
            cur_s[0] = _scan_block(lambda j: buf0[i, pl.ds(j * 16, 16)], F0 // 16, 0)

            def fb_body(blk, carry):
                @pl.when(cur_s[0] >= BIG)
                def _():
                    pltpu.sync_copy(
                        mask_hbm.at[row_base + i, pl.ds(blk * FB, FB)], buf_fb
                    )
                    bmin = _scan_block(
                        lambda j: buf_fb[pl.ds(j * 16, 16)], FB // 16, blk * FB
                    )
                    cur_s[0] = jnp.minimum(cur_s[0], bmin)
                return carry

            lax.fori_loop(0, NBLK, fb_body, jnp.int32(0), unroll=False)
            s0 = cur_s[0]
            action = jnp.where(s0 >= BIG, jnp.int32(0), s0)
            res_vec = jnp.where(lane == i, action, res_vec)

        out_v[...] = res_vec
        pltpu.sync_copy(out_v, out_hbm.at[wid])

    return full_kernel


_get_head_kernel = functools.cache(_make_head_kernel)
_get_full_kernel = functools.cache(_make_full_kernel)


def kernel(allowed_action_mask):
    head32 = allowed_action_mask[:, :F0].astype(jnp.int32)
    out1 = _get_head_kernel()(head32)
    actions = out1[:, :ROWS_PER_W].reshape(ROWS)
    all_found = jnp.all(out1[:, ROWS_PER_W:2 * ROWS_PER_W] != 0)

    def _tier2():
        full32 = allowed_action_mask.astype(jnp.int32)
        out2 = _get_full_kernel()(full32)
        return out2[:, :ROWS_PER_W].reshape(ROWS)

    return lax.cond(all_found, lambda: actions, _tier2)


# speedup vs baseline: 3.6236x; 1.1056x over previous
import functools

import jax
import jax.numpy as jnp
from jax import lax
from jax.experimental import pallas as pl
from jax.experimental.pallas import tpu as pltpu
from jax.experimental.pallas import tpu_sc as plsc

ROWS = 128
COLS = 32768
NW = 16
ROWS_PER_W = ROWS // NW
F0 = 512
FB = 2048
NBLK = COLS // FB
BIG = 1 << 30

_GDN = lax.GatherDimensionNumbers(
    offset_dims=(), collapsed_slice_dims=(0,), start_index_map=(0,)
)


def _shuffle(v, perm):
    return lax.gather(
        v, perm[:, None], dimension_numbers=_GDN, slice_sizes=(1,),
        mode=lax.GatherScatterMode.PROMISE_IN_BOUNDS,
    )


def _scan_block(get_vec, num_vecs, col_base):
    lane = lax.iota(jnp.int32, 16)
    acc = jnp.full((16,), BIG, jnp.int32)
    for j in range(num_vecs):
        v = get_vec(j)
        cand = lane + (col_base + j * 16)
        acc = jnp.minimum(acc, jnp.where(v != 0, cand, BIG))
    for s in (8, 4, 2, 1):
        acc = jnp.minimum(acc, _shuffle(acc, lane ^ s))
    return acc[0]


def _mesh():
    return plsc.VectorSubcoreMesh(
        core_axis_name="c", subcore_axis_name="s", num_cores=1, num_subcores=16
    )


def _make_head_kernel():
    @functools.partial(
        pl.kernel,
        out_type=(
            jax.ShapeDtypeStruct((ROWS,), jnp.int32),
            jax.ShapeDtypeStruct((NW, 16), jnp.int32),
        ),
        mesh=_mesh(),
        scratch_types=[
            pltpu.VMEM((ROWS_PER_W, F0), jnp.int32),
            pltpu.VMEM((16,), jnp.int32),
            pltpu.VMEM((16,), jnp.int32),
        ],
    )
    def head_kernel(head_hbm, act_hbm, flag_hbm, buf0, act_v, flag_v):
        wid = lax.axis_index("s")
        row_base = wid * ROWS_PER_W
        pltpu.sync_copy(head_hbm.at[pl.ds(row_base, ROWS_PER_W), pl.ds(0, F0)], buf0)

        lane = lax.iota(jnp.int32, 16)
        act_vec = jnp.zeros((16,), jnp.int32)
        flag_vec = jnp.ones((16,), jnp.int32)
        for i in range(ROWS_PER_W):
            s0 = _scan_block(lambda j: buf0[i, pl.ds(j * 16, 16)], F0 // 16, 0)
            action = jnp.where(s0 < BIG, s0, jnp.int32(0))
            act_vec = jnp.where(lane == i, action, act_vec)
            flag_vec = jnp.where(lane == i, (s0 < BIG).astype(jnp.int32), flag_vec)

        act_v[...] = act_vec
        flag_v[...] = flag_vec
        pltpu.sync_copy(act_v.at[pl.ds(0, ROWS_PER_W)], act_hbm.at[pl.ds(row_base, ROWS_PER_W)])
        pltpu.sync_copy(flag_v, flag_hbm.at[wid])

    return head_kernel


def _make_full_kernel():
    @functools.partial(
        pl.kernel,
        out_type=jax.ShapeDtypeStruct((ROWS,), jnp.int32),
        mesh=_mesh(),
        scratch_types=[
            pltpu.VMEM((ROWS_PER_W, F0), jnp.int32),
            pltpu.VMEM((FB,), jnp.int32),
            pltpu.VMEM((16,), jnp.int32),
            pltpu.SMEM((1,), jnp.int32),
        ],
    )
    def full_kernel(mask_hbm, act_hbm, buf0, buf_fb, act_v, cur_s):
        wid = lax.axis_index("s")
        row_base = wid * ROWS_PER_W
        pltpu.sync_copy(mask_hbm.at[pl.ds(row_base, ROWS_PER_W), pl.ds(0, F0)], buf0)

        lane = lax.iota(jnp.int32, 16)
        act_vec = jnp.zeros((16,), jnp.int32)
        for i in range(ROWS_PER_W):
            cur_s[0] = _scan_block(lambda j: buf0[i, pl.ds(j * 16, 16)], F0 // 16, 0)

            def fb_body(blk, carry):
                @pl.when(cur_s[0] >= BIG)
                def _():
                    pltpu.sync_copy(
                        mask_hbm.at[row_base + i, pl.ds(blk * FB, FB)], buf_fb
                    )
                    bmin = _scan_block(
                        lambda j: buf_fb[pl.ds(j * 16, 16)], FB // 16, blk * FB
                    )
                    cur_s[0] = jnp.minimum(cur_s[0], bmin)
                return carry

            lax.fori_loop(0, NBLK, fb_body, jnp.int32(0), unroll=False)
            s0 = cur_s[0]
            action = jnp.where(s0 >= BIG, jnp.int32(0), s0)
            act_vec = jnp.where(lane == i, action, act_vec)

        act_v[...] = act_vec
        pltpu.sync_copy(act_v.at[pl.ds(0, ROWS_PER_W)], act_hbm.at[pl.ds(row_base, ROWS_PER_W)])

    return full_kernel


_get_head_kernel = functools.cache(_make_head_kernel)
_get_full_kernel = functools.cache(_make_full_kernel)


def kernel(allowed_action_mask):
    head32 = allowed_action_mask[:, :F0].astype(jnp.int32)
    actions, flags = _get_head_kernel()(head32)
    all_found = jnp.all(flags != 0)

    def _tier2():
        full32 = allowed_action_mask.astype(jnp.int32)
        return _get_full_kernel()(full32)

    return lax.cond(all_found, lambda: actions, _tier2)
